# baseline (speedup 1.0000x reference)
import functools

import jax
import jax.numpy as jnp
from jax import lax
from jax.experimental import pallas as pl
from jax.experimental.pallas import tpu as pltpu
from jax.experimental.pallas import tpu_sc as plsc

N = 10000
E = 320000
D = 128

NC = 2
NS = 16
NW = NC * NS
CHUNK = 128
EPW = E // NW
NFULL = EPW // CHUNK
TAIL = EPW - NFULL * CHUNK
NCHUNK = 80
E_PAD = CHUNK * NCHUNK * NW
N_PAD = 10240
ROWS_PER_TILE = N_PAD // NS
RB = ROWS_PER_TILE // CHUNK


_MESH = plsc.VectorSubcoreMesh(core_axis_name="c", subcore_axis_name="s")


def _make_agg():
    scratch = [
        pltpu.VMEM((EPW,), jnp.int32),
        pltpu.VMEM((CHUNK,), jnp.int32),
        pltpu.VMEM((CHUNK,), jnp.int32),
        pltpu.VMEM((CHUNK, D), jnp.float32),
        pltpu.VMEM((CHUNK, D), jnp.float32),
        pltpu.VMEM((TAIL, D), jnp.float32),
        pltpu.VMEM((TAIL,), jnp.int32),
        pltpu.VMEM_SHARED((N_PAD, D), jnp.float32),
        pltpu.SemaphoreType.DMA,
        pltpu.SemaphoreType.DMA,
        pltpu.SemaphoreType.DMA,
        pltpu.SemaphoreType.DMA,
        pltpu.SemaphoreType.DMA,
        pltpu.SemaphoreType.DMA,
    ]

    def body(h_hbm, src_hbm, dst_hbm, out_hbm, src_v, dst0, dst1,
             rows0, rows1, rows_t, dst_t, acc, g0, g1, d0, d1, s0, s1):
        c = lax.axis_index("c")
        s = lax.axis_index("s")
        wid = s * NC + c
        e0 = wid * EPW

        zv = jnp.zeros((16,), jnp.float32)
        def fill_body(i, _):
            r = i // 8
            cc = (i % 8) * 16
            rows0[r, pl.ds(cc, 16)] = zv
            return 0
        lax.fori_loop(0, CHUNK * 8, fill_body, 0)

        pltpu.sync_copy(src_hbm.at[pl.ds(e0, EPW)], src_v)

        def zero_body(j, _):
            r0 = s * ROWS_PER_TILE + j * CHUNK
            pltpu.sync_copy(rows0, acc.at[pl.ds(r0, CHUNK)])
            return 0
        lax.fori_loop(0, RB, zero_body, 0)
        plsc.subcore_barrier()

        def sidx(j):
            return src_v.at[pl.ds(j * CHUNK, CHUNK)]

        def issue_g(b_rows, b_sem, j):
            pltpu.async_copy(h_hbm.at[sidx(j)], b_rows, b_sem)

        def wait_g(b_rows, b_sem, j):
            pltpu.make_async_copy(h_hbm.at[sidx(j)], b_rows, b_sem).wait()

        def issue_d(b_dst, b_sem, j):
            pltpu.async_copy(dst_hbm.at[pl.ds(e0 + j * CHUNK, CHUNK)],
                             b_dst, b_sem)

        def wait_d(b_dst, b_sem, j):
            pltpu.make_async_copy(dst_hbm.at[pl.ds(e0 + j * CHUNK, CHUNK)],
                                  b_dst, b_sem).wait()

        def wait_s(b_rows, b_dst, b_sem):
            pltpu.make_async_copy(b_rows, acc.at[b_dst], b_sem).wait()

        issue_g(rows0, g0, 0)
        issue_d(dst0, d0, 0)
        wait_g(rows0, g0, 0)
        wait_d(dst0, d0, 0)
        pltpu.async_copy(rows0, acc.at[dst0], s0, add=True)
        issue_g(rows1, g1, 1)
        issue_d(dst1, d1, 1)

        def pair_body(jp, _):
            j = 2 * jp + 1
            wait_g(rows1, g1, j)
            wait_d(dst1, d1, j)
            pltpu.async_copy(rows1, acc.at[dst1], s1, add=True)
            wait_s(rows0, dst0, s0)
            issue_g(rows0, g0, j + 1)
            issue_d(dst0, d0, j + 1)
            wait_g(rows0, g0, j + 1)
            wait_d(dst0, d0, j + 1)
            pltpu.async_copy(rows0, acc.at[dst0], s0, add=True)
            wait_s(rows1, dst1, s1)
            issue_g(rows1, g1, j + 2)
            issue_d(dst1, d1, j + 2)
            return 0
        lax.fori_loop(0, NFULL // 2 - 1, pair_body, 0)

        jl = NFULL - 1
        wait_g(rows1, g1, jl)
        wait_d(dst1, d1, jl)
        pltpu.async_copy(rows1, acc.at[dst1], s1, add=True)
        t0 = NFULL * CHUNK
        pltpu.async_copy(h_hbm.at[src_v.at[pl.ds(t0, TAIL)]], rows_t, g0)
        pltpu.async_copy(dst_hbm.at[pl.ds(e0 + t0, TAIL)], dst_t, d0)
        pltpu.make_async_copy(h_hbm.at[src_v.at[pl.ds(t0, TAIL)]], rows_t,
                              g0).wait()
        pltpu.make_async_copy(dst_hbm.at[pl.ds(e0 + t0, TAIL)], dst_t,
                              d0).wait()
        pltpu.sync_copy(rows_t, acc.at[dst_t], add=True)
        wait_s(rows0, dst0, s0)
        wait_s(rows1, dst1, s1)
        plsc.subcore_barrier()

        def rb_body(j, _):
            r0 = s * ROWS_PER_TILE + j * CHUNK
            pltpu.sync_copy(acc.at[pl.ds(r0, CHUNK)],
                            out_hbm.at[pl.ds(c * N_PAD + r0, CHUNK)])
            return 0
        lax.fori_loop(0, RB, rb_body, 0)

    return pl.kernel(body, mesh=_MESH,
                     out_type=jax.ShapeDtypeStruct((NC * N_PAD, D),
                                                   jnp.float32),
                     scratch_types=scratch)


def _make_cnt():
    scratch = [
        pltpu.VMEM((NCHUNK, CHUNK), jnp.int32),
        pltpu.VMEM((CHUNK,), jnp.float32),
        pltpu.VMEM((CHUNK,), jnp.float32),
        pltpu.VMEM_SHARED((N_PAD,), jnp.float32),
    ]

    def body(dst_hbm, cnt_hbm, dst_v, ones_v, z_v, cnt_acc):
        c = lax.axis_index("c")
        s = lax.axis_index("s")
        wid = s * NC + c

        zv = jnp.zeros((16,), jnp.float32)
        ov = jnp.ones((16,), jnp.float32)
        def fill_body(i, _):
            ones_v[pl.ds(i * 16, 16)] = ov
            z_v[pl.ds(i * 16, 16)] = zv
            return 0
        lax.fori_loop(0, CHUNK // 16, fill_body, 0)

        pltpu.sync_copy(dst_hbm.at[pl.ds(wid * NCHUNK, NCHUNK)], dst_v)

        def zero_body(j, _):
            r0 = s * ROWS_PER_TILE + j * CHUNK
            pltpu.sync_copy(z_v, cnt_acc.at[pl.ds(r0, CHUNK)])
            return 0
        lax.fori_loop(0, RB, zero_body, 0)
        plsc.subcore_barrier()

        def edge_body(j, _):
            pltpu.sync_copy(ones_v, cnt_acc.at[dst_v.at[j]], add=True)
            return 0
        lax.fori_loop(0, NCHUNK, edge_body, 0)
        plsc.subcore_barrier()

        def rb_body(j, _):
            r0 = s * ROWS_PER_TILE + j * CHUNK
            pltpu.sync_copy(cnt_acc.at[pl.ds(r0, CHUNK)],
                            cnt_hbm.at[pl.ds(c * N_PAD + r0, CHUNK)])
            return 0
        lax.fori_loop(0, RB, rb_body, 0)

    return pl.kernel(body, mesh=_MESH,
                     out_type=jax.ShapeDtypeStruct((NC * N_PAD,),
                                                   jnp.float32),
                     scratch_types=scratch)


_agg = _make_agg()
_cnt = _make_cnt()


def _tc_self(h, Ws, b):
    B = 512

    def body(h_ref, ws_ref, b_ref, o_ref):
        acc = jnp.dot(h_ref[...], ws_ref[...],
                      preferred_element_type=jnp.float32)
        o_ref[...] = acc + b_ref[...]

    return pl.pallas_call(
        body,
        grid=(N_PAD // B,),
        in_specs=[
            pl.BlockSpec((B, D), lambda i: (i, 0)),
            pl.BlockSpec((D, D), lambda i: (0, 0)),
            pl.BlockSpec((1, D), lambda i: (0, 0)),
        ],
        out_specs=pl.BlockSpec((B, D), lambda i: (i, 0)),
        out_shape=jax.ShapeDtypeStruct((N_PAD, D), jnp.float32),
    )(h, Ws, b.reshape(1, D))


def _tc_comb(p, cnt, sself, Wn, relu):
    B = 512

    def body(p0_ref, p1_ref, c0_ref, c1_ref, s_ref, wn_ref, o_ref):
        cnt_col = c0_ref[...] + c1_ref[...]
        inv = 1.0 / jnp.maximum(cnt_col, 1.0)
        mean = (p0_ref[...] + p1_ref[...]) * inv
        acc = jnp.dot(mean, wn_ref[...], preferred_element_type=jnp.float32)
        acc = acc + s_ref[...]
        o_ref[...] = jnp.maximum(acc, 0.0) if relu else acc

    nb = N_PAD // B
    return pl.pallas_call(
        body,
        grid=(N_PAD // B,),
        in_specs=[
            pl.BlockSpec((B, D), lambda i: (i, 0)),
            pl.BlockSpec((B, D), lambda i, _nb=nb: (i + _nb, 0)),
            pl.BlockSpec((B, 1), lambda i: (i, 0)),
            pl.BlockSpec((B, 1), lambda i, _nb=nb: (i + _nb, 0)),
            pl.BlockSpec((B, D), lambda i: (i, 0)),
            pl.BlockSpec((D, D), lambda i: (0, 0)),
        ],
        out_specs=pl.BlockSpec((B, D), lambda i: (i, 0)),
        out_shape=jax.ShapeDtypeStruct((N_PAD, D), jnp.float32),
    )(p, p, cnt.reshape(NC * N_PAD, 1), cnt.reshape(NC * N_PAD, 1), sself,
      Wn)


def kernel(x, edge_index, Wn0, Ws0, b0, Wn1, Ws1, b1, Wn2, Ws2, b2):
    src = edge_index[0]
    dst = edge_index[1]
    pad = E_PAD - E
    pad_dst = N + jnp.arange(pad, dtype=jnp.int32) % (N_PAD - N)
    dst_p = jnp.concatenate([dst, pad_dst]).reshape(NW * NCHUNK, CHUNK)
    xp = jnp.pad(x, ((0, N_PAD - N), (0, 0)))

    cnt = _cnt(dst_p)
    s0 = _tc_self(xp, Ws0, b0)
    p = _agg(xp, src, dst)
    h1 = _tc_comb(p, cnt, s0, Wn0, True)
    s1 = _tc_self(h1, Ws1, b1)
    p = _agg(h1, src, dst)
    h2 = _tc_comb(p, cnt, s1, Wn1, True)
    s2 = _tc_self(h2, Ws2, b2)
    p = _agg(h2, src, dst)
    h3 = _tc_comb(p, cnt, s2, Wn2, False)
    return h3[:N]


# speedup vs baseline: 1.0376x; 1.0376x over previous
import functools

import jax
import jax.numpy as jnp
from jax import lax
from jax.experimental import pallas as pl
from jax.experimental.pallas import tpu as pltpu
from jax.experimental.pallas import tpu_sc as plsc

N = 10000
E = 320000
D = 128

NC = 2
NS = 16
NW = NC * NS
CHUNK = 128
EPW = E // NW
NFULL = EPW // CHUNK
TAIL = EPW - NFULL * CHUNK
NCHUNK = 80
E_PAD = CHUNK * NCHUNK * NW
N_PAD = 10240
ROWS_PER_TILE = N_PAD // NS
RB = ROWS_PER_TILE // CHUNK


_MESH = plsc.VectorSubcoreMesh(core_axis_name="c", subcore_axis_name="s")


def _make_agg():
    scratch = [
        pltpu.VMEM((EPW,), jnp.int32),
        pltpu.VMEM((CHUNK,), jnp.int32),
        pltpu.VMEM((CHUNK,), jnp.int32),
        pltpu.VMEM((CHUNK, D), jnp.float32),
        pltpu.VMEM((CHUNK, D), jnp.float32),
        pltpu.VMEM((TAIL, D), jnp.float32),
        pltpu.VMEM((TAIL,), jnp.int32),
        pltpu.VMEM_SHARED((N_PAD, D), jnp.float32),
        pltpu.SemaphoreType.DMA,
        pltpu.SemaphoreType.DMA,
        pltpu.SemaphoreType.DMA,
        pltpu.SemaphoreType.DMA,
        pltpu.SemaphoreType.DMA,
        pltpu.SemaphoreType.DMA,
    ]

    def body(h_hbm, src_hbm, dst_hbm, out_hbm, src_v, dst0, dst1,
             rows0, rows1, rows_t, dst_t, acc, g0, g1, d0, d1, s0, s1):
        c = lax.axis_index("c")
        s = lax.axis_index("s")
        wid = s * NC + c
        e0 = wid * EPW

        zv = jnp.zeros((16,), jnp.float32)
        def fill_body(i, _):
            r = i // 8
            cc = (i % 8) * 16
            rows0[r, pl.ds(cc, 16)] = zv
            return 0
        lax.fori_loop(0, CHUNK * 8, fill_body, 0)

        pltpu.sync_copy(src_hbm.at[pl.ds(e0, EPW)], src_v)

        def zero_body(j, _):
            r0 = s * ROWS_PER_TILE + j * CHUNK
            pltpu.sync_copy(rows0, acc.at[pl.ds(r0, CHUNK)])
            return 0
        lax.fori_loop(0, RB, zero_body, 0)
        plsc.subcore_barrier()

        def sidx(j):
            return src_v.at[pl.ds(j * CHUNK, CHUNK)]

        def issue_g(b_rows, b_sem, j):
            pltpu.async_copy(h_hbm.at[sidx(j)], b_rows, b_sem)

        def wait_g(b_rows, b_sem, j):
            pltpu.make_async_copy(h_hbm.at[sidx(j)], b_rows, b_sem).wait()

        def issue_d(b_dst, b_sem, j):
            pltpu.async_copy(dst_hbm.at[pl.ds(e0 + j * CHUNK, CHUNK)],
                             b_dst, b_sem)

        def wait_d(b_dst, b_sem, j):
            pltpu.make_async_copy(dst_hbm.at[pl.ds(e0 + j * CHUNK, CHUNK)],
                                  b_dst, b_sem).wait()

        def wait_s(b_rows, b_dst, b_sem):
            pltpu.make_async_copy(b_rows, acc.at[b_dst], b_sem).wait()

        issue_g(rows0, g0, 0)
        issue_d(dst0, d0, 0)
        wait_g(rows0, g0, 0)
        wait_d(dst0, d0, 0)
        pltpu.async_copy(rows0, acc.at[dst0], s0, add=True)
        issue_g(rows1, g1, 1)
        issue_d(dst1, d1, 1)

        def pair_body(jp, _):
            j = 2 * jp + 1
            wait_g(rows1, g1, j)
            wait_d(dst1, d1, j)
            pltpu.async_copy(rows1, acc.at[dst1], s1, add=True)
            wait_s(rows0, dst0, s0)
            issue_g(rows0, g0, j + 1)
            issue_d(dst0, d0, j + 1)
            wait_g(rows0, g0, j + 1)
            wait_d(dst0, d0, j + 1)
            pltpu.async_copy(rows0, acc.at[dst0], s0, add=True)
            wait_s(rows1, dst1, s1)
            issue_g(rows1, g1, j + 2)
            issue_d(dst1, d1, j + 2)
            return 0
        lax.fori_loop(0, NFULL // 2 - 1, pair_body, 0)

        jl = NFULL - 1
        wait_g(rows1, g1, jl)
        wait_d(dst1, d1, jl)
        pltpu.async_copy(rows1, acc.at[dst1], s1, add=True)
        t0 = NFULL * CHUNK
        pltpu.async_copy(h_hbm.at[src_v.at[pl.ds(t0, TAIL)]], rows_t, g0)
        pltpu.async_copy(dst_hbm.at[pl.ds(e0 + t0, TAIL)], dst_t, d0)
        pltpu.make_async_copy(h_hbm.at[src_v.at[pl.ds(t0, TAIL)]], rows_t,
                              g0).wait()
        pltpu.make_async_copy(dst_hbm.at[pl.ds(e0 + t0, TAIL)], dst_t,
                              d0).wait()
        pltpu.sync_copy(rows_t, acc.at[dst_t], add=True)
        wait_s(rows0, dst0, s0)
        wait_s(rows1, dst1, s1)
        plsc.subcore_barrier()

        def rb_body(j, _):
            r0 = s * ROWS_PER_TILE + j * CHUNK
            pltpu.sync_copy(acc.at[pl.ds(r0, CHUNK)],
                            out_hbm.at[pl.ds(c * N_PAD + r0, CHUNK)])
            return 0
        lax.fori_loop(0, RB, rb_body, 0)

    return pl.kernel(body, mesh=_MESH,
                     out_type=jax.ShapeDtypeStruct((NC * N_PAD, D),
                                                   jnp.float32),
                     scratch_types=scratch)


def _make_cnt():
    scratch = [
        pltpu.VMEM((NCHUNK, CHUNK), jnp.int32),
        pltpu.VMEM((CHUNK,), jnp.float32),
        pltpu.VMEM((CHUNK,), jnp.float32),
        pltpu.VMEM_SHARED((N_PAD,), jnp.float32),
    ]

    def body(dst_hbm, cnt_hbm, dst_v, ones_v, z_v, cnt_acc):
        c = lax.axis_index("c")
        s = lax.axis_index("s")
        wid = s * NC + c

        zv = jnp.zeros((16,), jnp.float32)
        ov = jnp.ones((16,), jnp.float32)
        def fill_body(i, _):
            ones_v[pl.ds(i * 16, 16)] = ov
            z_v[pl.ds(i * 16, 16)] = zv
            return 0
        lax.fori_loop(0, CHUNK // 16, fill_body, 0)

        pltpu.sync_copy(dst_hbm.at[pl.ds(wid * NCHUNK, NCHUNK)], dst_v)

        def zero_body(j, _):
            r0 = s * ROWS_PER_TILE + j * CHUNK
            pltpu.sync_copy(z_v, cnt_acc.at[pl.ds(r0, CHUNK)])
            return 0
        lax.fori_loop(0, RB, zero_body, 0)
        plsc.subcore_barrier()

        def edge_body(j, _):
            pltpu.sync_copy(ones_v, cnt_acc.at[dst_v.at[j]], add=True)
            return 0
        lax.fori_loop(0, NCHUNK, edge_body, 0)
        plsc.subcore_barrier()

        def rb_body(j, _):
            r0 = s * ROWS_PER_TILE + j * CHUNK
            pltpu.sync_copy(cnt_acc.at[pl.ds(r0, CHUNK)],
                            cnt_hbm.at[pl.ds(c * N_PAD + r0, CHUNK)])
            return 0
        lax.fori_loop(0, RB, rb_body, 0)

    return pl.kernel(body, mesh=_MESH,
                     out_type=jax.ShapeDtypeStruct((NC * N_PAD,),
                                                   jnp.float32),
                     scratch_types=scratch)


_agg = _make_agg()
_cnt = _make_cnt()


def _tc_self(h, Ws, b):
    B = 1024

    def body(h_ref, ws_ref, b_ref, o_ref):
        acc = jnp.dot(h_ref[...], ws_ref[...],
                      preferred_element_type=jnp.float32)
        o_ref[...] = acc + b_ref[...]

    return pl.pallas_call(
        body,
        grid=(N_PAD // B,),
        in_specs=[
            pl.BlockSpec((B, D), lambda i: (i, 0)),
            pl.BlockSpec((D, D), lambda i: (0, 0)),
            pl.BlockSpec((1, D), lambda i: (0, 0)),
        ],
        out_specs=pl.BlockSpec((B, D), lambda i: (i, 0)),
        out_shape=jax.ShapeDtypeStruct((N_PAD, D), jnp.float32),
    )(h, Ws, b.reshape(1, D))


def _tc_comb(p, cnt, sself, Wn, relu):
    B = 1024

    def body(p0_ref, p1_ref, c0_ref, c1_ref, s_ref, wn_ref, o_ref):
        cnt_col = c0_ref[...] + c1_ref[...]
        inv = 1.0 / jnp.maximum(cnt_col, 1.0)
        mean = (p0_ref[...] + p1_ref[...]) * inv
        acc = jnp.dot(mean, wn_ref[...], preferred_element_type=jnp.float32)
        acc = acc + s_ref[...]
        o_ref[...] = jnp.maximum(acc, 0.0) if relu else acc

    nb = N_PAD // B
    return pl.pallas_call(
        body,
        grid=(N_PAD // B,),
        in_specs=[
            pl.BlockSpec((B, D), lambda i: (i, 0)),
            pl.BlockSpec((B, D), lambda i, _nb=nb: (i + _nb, 0)),
            pl.BlockSpec((B, 1), lambda i: (i, 0)),
            pl.BlockSpec((B, 1), lambda i, _nb=nb: (i + _nb, 0)),
            pl.BlockSpec((B, D), lambda i: (i, 0)),
            pl.BlockSpec((D, D), lambda i: (0, 0)),
        ],
        out_specs=pl.BlockSpec((B, D), lambda i: (i, 0)),
        out_shape=jax.ShapeDtypeStruct((N_PAD, D), jnp.float32),
    )(p, p, cnt.reshape(NC * N_PAD, 1), cnt.reshape(NC * N_PAD, 1), sself,
      Wn)


def kernel(x, edge_index, Wn0, Ws0, b0, Wn1, Ws1, b1, Wn2, Ws2, b2):
    src = edge_index[0]
    dst = edge_index[1]
    pad = E_PAD - E
    pad_dst = N + jnp.arange(pad, dtype=jnp.int32) % (N_PAD - N)
    dst_p = jnp.concatenate([dst, pad_dst]).reshape(NW * NCHUNK, CHUNK)
    xp = jnp.pad(x, ((0, N_PAD - N), (0, 0)))

    cnt = _cnt(dst_p)
    s0 = _tc_self(xp, Ws0, b0)
    p = _agg(xp, src, dst)
    h1 = _tc_comb(p, cnt, s0, Wn0, True)
    s1 = _tc_self(h1, Ws1, b1)
    p = _agg(h1, src, dst)
    h2 = _tc_comb(p, cnt, s1, Wn1, True)
    s2 = _tc_self(h2, Ws2, b2)
    p = _agg(h2, src, dst)
    h3 = _tc_comb(p, cnt, s2, Wn2, False)
    return h3[:N]


# speedup vs baseline: 1.0482x; 1.0103x over previous
import functools

import jax
import jax.numpy as jnp
from jax import lax
from jax.experimental import pallas as pl
from jax.experimental.pallas import tpu as pltpu
from jax.experimental.pallas import tpu_sc as plsc

N = 10000
E = 320000
D = 128

NC = 2
NS = 16
NW = NC * NS
CHUNK = 128
EPW = E // NW
NFULL = EPW // CHUNK
TAIL = EPW - NFULL * CHUNK
NCHUNK = 80
E_PAD = CHUNK * NCHUNK * NW
N_PAD = 10240
ROWS_PER_TILE = N_PAD // NS
RB = ROWS_PER_TILE // CHUNK


_MESH = plsc.VectorSubcoreMesh(core_axis_name="c", subcore_axis_name="s")


def _make_agg():
    scratch = [
        pltpu.VMEM((EPW,), jnp.int32),
        pltpu.VMEM((CHUNK,), jnp.int32),
        pltpu.VMEM((CHUNK,), jnp.int32),
        pltpu.VMEM((CHUNK, D), jnp.float32),
        pltpu.VMEM((CHUNK, D), jnp.float32),
        pltpu.VMEM((TAIL, D), jnp.float32),
        pltpu.VMEM((TAIL,), jnp.int32),
        pltpu.VMEM_SHARED((N_PAD, D), jnp.float32),
        pltpu.SemaphoreType.DMA,
        pltpu.SemaphoreType.DMA,
        pltpu.SemaphoreType.DMA,
        pltpu.SemaphoreType.DMA,
        pltpu.SemaphoreType.DMA,
        pltpu.SemaphoreType.DMA,
    ]

    def body(h_hbm, src_hbm, dst_hbm, out_hbm, src_v, dst0, dst1,
             rows0, rows1, rows_t, dst_t, acc, g0, g1, d0, d1, s0, s1):
        c = lax.axis_index("c")
        s = lax.axis_index("s")
        wid = s * NC + c
        e0 = wid * EPW

        zv = jnp.zeros((16,), jnp.float32)
        def fill_body(i, _):
            r = i // 8
            cc = (i % 8) * 16
            rows0[r, pl.ds(cc, 16)] = zv
            return 0
        lax.fori_loop(0, CHUNK * 8, fill_body, 0)

        pltpu.sync_copy(src_hbm.at[pl.ds(e0, EPW)], src_v)

        def zero_body(j, _):
            r0 = s * ROWS_PER_TILE + j * CHUNK
            pltpu.sync_copy(rows0, acc.at[pl.ds(r0, CHUNK)])
            return 0
        lax.fori_loop(0, RB, zero_body, 0)
        plsc.subcore_barrier()

        def sidx(j):
            return src_v.at[pl.ds(j * CHUNK, CHUNK)]

        def issue_g(b_rows, b_sem, j):
            pltpu.async_copy(h_hbm.at[sidx(j)], b_rows, b_sem)

        def wait_g(b_rows, b_sem, j):
            pltpu.make_async_copy(h_hbm.at[sidx(j)], b_rows, b_sem).wait()

        def issue_d(b_dst, b_sem, j):
            pltpu.async_copy(dst_hbm.at[pl.ds(e0 + j * CHUNK, CHUNK)],
                             b_dst, b_sem)

        def wait_d(b_dst, b_sem, j):
            pltpu.make_async_copy(dst_hbm.at[pl.ds(e0 + j * CHUNK, CHUNK)],
                                  b_dst, b_sem).wait()

        def wait_s(b_rows, b_dst, b_sem):
            pltpu.make_async_copy(b_rows, acc.at[b_dst], b_sem).wait()

        issue_g(rows0, g0, 0)
        issue_d(dst0, d0, 0)
        wait_g(rows0, g0, 0)
        wait_d(dst0, d0, 0)
        pltpu.async_copy(rows0, acc.at[dst0], s0, add=True)
        issue_g(rows1, g1, 1)
        issue_d(dst1, d1, 1)

        def pair_body(jp, _):
            j = 2 * jp + 1
            wait_g(rows1, g1, j)
            wait_d(dst1, d1, j)
            pltpu.async_copy(rows1, acc.at[dst1], s1, add=True)
            wait_s(rows0, dst0, s0)
            issue_g(rows0, g0, j + 1)
            issue_d(dst0, d0, j + 1)
            wait_g(rows0, g0, j + 1)
            wait_d(dst0, d0, j + 1)
            pltpu.async_copy(rows0, acc.at[dst0], s0, add=True)
            wait_s(rows1, dst1, s1)
            issue_g(rows1, g1, j + 2)
            issue_d(dst1, d1, j + 2)
            return 0
        lax.fori_loop(0, NFULL // 2 - 1, pair_body, 0)

        jl = NFULL - 1
        wait_g(rows1, g1, jl)
        wait_d(dst1, d1, jl)
        pltpu.async_copy(rows1, acc.at[dst1], s1, add=True)
        t0 = NFULL * CHUNK
        pltpu.async_copy(h_hbm.at[src_v.at[pl.ds(t0, TAIL)]], rows_t, g0)
        pltpu.async_copy(dst_hbm.at[pl.ds(e0 + t0, TAIL)], dst_t, d0)
        pltpu.make_async_copy(h_hbm.at[src_v.at[pl.ds(t0, TAIL)]], rows_t,
                              g0).wait()
        pltpu.make_async_copy(dst_hbm.at[pl.ds(e0 + t0, TAIL)], dst_t,
                              d0).wait()
        pltpu.sync_copy(rows_t, acc.at[dst_t], add=True)
        wait_s(rows0, dst0, s0)
        wait_s(rows1, dst1, s1)
        plsc.subcore_barrier()

        def rb_body(j, _):
            r0 = s * ROWS_PER_TILE + j * CHUNK
            pltpu.sync_copy(acc.at[pl.ds(r0, CHUNK)],
                            out_hbm.at[pl.ds(c * N_PAD + r0, CHUNK)])
            return 0
        lax.fori_loop(0, RB, rb_body, 0)

    return pl.kernel(body, mesh=_MESH,
                     out_type=jax.ShapeDtypeStruct((NC * N_PAD, D),
                                                   jnp.float32),
                     scratch_types=scratch)


def _make_cnt():
    scratch = [
        pltpu.VMEM((NCHUNK, CHUNK), jnp.int32),
        pltpu.VMEM((CHUNK,), jnp.float32),
        pltpu.VMEM((CHUNK,), jnp.float32),
        pltpu.VMEM_SHARED((N_PAD,), jnp.float32),
    ]

    def body(dst_hbm, cnt_hbm, dst_v, ones_v, z_v, cnt_acc):
        c = lax.axis_index("c")
        s = lax.axis_index("s")
        wid = s * NC + c

        zv = jnp.zeros((16,), jnp.float32)
        ov = jnp.ones((16,), jnp.float32)
        def fill_body(i, _):
            ones_v[pl.ds(i * 16, 16)] = ov
            z_v[pl.ds(i * 16, 16)] = zv
            return 0
        lax.fori_loop(0, CHUNK // 16, fill_body, 0)

        pltpu.sync_copy(dst_hbm.at[pl.ds(wid * NCHUNK, NCHUNK)], dst_v)

        def zero_body(j, _):
            r0 = s * ROWS_PER_TILE + j * CHUNK
            pltpu.sync_copy(z_v, cnt_acc.at[pl.ds(r0, CHUNK)])
            return 0
        lax.fori_loop(0, RB, zero_body, 0)
        plsc.subcore_barrier()

        def edge_body(j, _):
            pltpu.sync_copy(ones_v, cnt_acc.at[dst_v.at[j]], add=True)
            return 0
        lax.fori_loop(0, NCHUNK, edge_body, 0)
        plsc.subcore_barrier()

        def rb_body(j, _):
            r0 = s * ROWS_PER_TILE + j * CHUNK
            pltpu.sync_copy(cnt_acc.at[pl.ds(r0, CHUNK)],
                            cnt_hbm.at[pl.ds(c * N_PAD + r0, CHUNK)])
            return 0
        lax.fori_loop(0, RB, rb_body, 0)

    return pl.kernel(body, mesh=_MESH,
                     out_type=jax.ShapeDtypeStruct((NC * N_PAD,),
                                                   jnp.float32),
                     scratch_types=scratch)


_agg = _make_agg()
_cnt = _make_cnt()


def _tc_self(h, Ws, b):
    B = 2048

    def body(h_ref, ws_ref, b_ref, o_ref):
        acc = jnp.dot(h_ref[...], ws_ref[...],
                      preferred_element_type=jnp.float32)
        o_ref[...] = acc + b_ref[...]

    return pl.pallas_call(
        body,
        grid=(N_PAD // B,),
        in_specs=[
            pl.BlockSpec((B, D), lambda i: (i, 0)),
            pl.BlockSpec((D, D), lambda i: (0, 0)),
            pl.BlockSpec((1, D), lambda i: (0, 0)),
        ],
        out_specs=pl.BlockSpec((B, D), lambda i: (i, 0)),
        out_shape=jax.ShapeDtypeStruct((N_PAD, D), jnp.float32),
    )(h, Ws, b.reshape(1, D))


def _tc_comb(p, cnt, sself, Wn, relu):
    B = 2048

    def body(p0_ref, p1_ref, c0_ref, c1_ref, s_ref, wn_ref, o_ref):
        cnt_col = c0_ref[...] + c1_ref[...]
        inv = 1.0 / jnp.maximum(cnt_col, 1.0)
        mean = (p0_ref[...] + p1_ref[...]) * inv
        acc = jnp.dot(mean, wn_ref[...], preferred_element_type=jnp.float32)
        acc = acc + s_ref[...]
        o_ref[...] = jnp.maximum(acc, 0.0) if relu else acc

    nb = N_PAD // B
    return pl.pallas_call(
        body,
        grid=(N_PAD // B,),
        in_specs=[
            pl.BlockSpec((B, D), lambda i: (i, 0)),
            pl.BlockSpec((B, D), lambda i, _nb=nb: (i + _nb, 0)),
            pl.BlockSpec((B, 1), lambda i: (i, 0)),
            pl.BlockSpec((B, 1), lambda i, _nb=nb: (i + _nb, 0)),
            pl.BlockSpec((B, D), lambda i: (i, 0)),
            pl.BlockSpec((D, D), lambda i: (0, 0)),
        ],
        out_specs=pl.BlockSpec((B, D), lambda i: (i, 0)),
        out_shape=jax.ShapeDtypeStruct((N_PAD, D), jnp.float32),
    )(p, p, cnt.reshape(NC * N_PAD, 1), cnt.reshape(NC * N_PAD, 1), sself,
      Wn)


def kernel(x, edge_index, Wn0, Ws0, b0, Wn1, Ws1, b1, Wn2, Ws2, b2):
    src = edge_index[0]
    dst = edge_index[1]
    pad = E_PAD - E
    pad_dst = N + jnp.arange(pad, dtype=jnp.int32) % (N_PAD - N)
    dst_p = jnp.concatenate([dst, pad_dst]).reshape(NW * NCHUNK, CHUNK)
    xp = jnp.pad(x, ((0, N_PAD - N), (0, 0)))

    cnt = _cnt(dst_p)
    s0 = _tc_self(xp, Ws0, b0)
    p = _agg(xp, src, dst)
    h1 = _tc_comb(p, cnt, s0, Wn0, True)
    s1 = _tc_self(h1, Ws1, b1)
    p = _agg(h1, src, dst)
    h2 = _tc_comb(p, cnt, s1, Wn1, True)
    s2 = _tc_self(h2, Ws2, b2)
    p = _agg(h2, src, dst)
    h3 = _tc_comb(p, cnt, s2, Wn2, False)
    return h3[:N]
